# baseline (speedup 1.0000x reference)
import functools

import jax
import jax.numpy as jnp
from jax import lax
from jax.experimental import pallas as pl
from jax.experimental.pallas import tpu as pltpu
from jax.experimental.pallas import tpu_sc as plsc

P = 100000
H = 128
B = 1024

_NC = 2
_NS = 16
_NW = _NC * _NS
_NIDX = 2 * B
_BPW = _NIDX // _NW

_PB = 2048
_NP = -(-P // _PB)


def _sc_gather(table, idx_flat):
    mesh = plsc.VectorSubcoreMesh(core_axis_name="c", subcore_axis_name="s")

    @functools.partial(
        pl.kernel,
        mesh=mesh,
        out_type=jax.ShapeDtypeStruct((_NIDX, H), jnp.float32),
        scratch_types=[
            pltpu.VMEM((_BPW,), jnp.int32),
            pltpu.VMEM((_BPW, H), jnp.float32),
            pltpu.SemaphoreType.DMA,
        ],
    )
    def k(table_hbm, idx_hbm, out_hbm, idx_v, rows_v, sem):
        wid = lax.axis_index("s") * _NC + lax.axis_index("c")
        base = wid * _BPW
        pltpu.sync_copy(idx_hbm.at[pl.ds(base, _BPW)], idx_v)
        pltpu.async_copy(table_hbm.at[idx_v], rows_v, sem).wait()
        pltpu.sync_copy(rows_v, out_hbm.at[pl.ds(base, _BPW)])

    return k(table, idx_flat)


def _mlp_body(emb_ref, w1_ref, b1_ref, w2_ref, b2_ref, w3_ref, b3_ref,
              out_ref, h2_ref):
    @pl.when(pl.program_id(0) == 0)
    def _():
        h1 = jnp.maximum(
            jnp.dot(emb_ref[...], w1_ref[...],
                    preferred_element_type=jnp.float32) + b1_ref[...], 0.0)
        h2_ref[...] = jnp.maximum(
            jnp.dot(h1, w2_ref[...],
                    preferred_element_type=jnp.float32) + b2_ref[...], 0.0)
    out_ref[...] = jnp.dot(h2_ref[...], w3_ref[...],
                           preferred_element_type=jnp.float32) + b3_ref[...]


def _mlp_tc(emb, W1, b1, W2, b2, W3, b3):
    return pl.pallas_call(
        _mlp_body,
        grid=(_NP,),
        in_specs=[
            pl.BlockSpec((B, 2 * H), lambda i: (0, 0)),
            pl.BlockSpec((2 * H, H), lambda i: (0, 0)),
            pl.BlockSpec((1, H), lambda i: (0, 0)),
            pl.BlockSpec((H, H), lambda i: (0, 0)),
            pl.BlockSpec((1, H), lambda i: (0, 0)),
            pl.BlockSpec((H, _PB), lambda i: (0, i)),
            pl.BlockSpec((1, _PB), lambda i: (0, i)),
        ],
        out_specs=pl.BlockSpec((B, _PB), lambda i: (0, i)),
        out_shape=jax.ShapeDtypeStruct((B, P), jnp.float32),
        scratch_shapes=[pltpu.VMEM((B, H), jnp.float32)],
    )(emb, W1, b1.reshape(1, H), W2, b2.reshape(1, H), W3,
      b3.reshape(1, P))


def kernel(x, table, W1, b1, W2, b2, W3, b3):
    idx_flat = x.reshape(-1).astype(jnp.int32)
    emb = _sc_gather(table, idx_flat).reshape(B, 2 * H)
    return _mlp_tc(emb, W1, b1, W2, b2, W3, b3)
